# baseline (speedup 1.0000x reference)
import functools

import jax
import jax.numpy as jnp
from jax import lax
from jax.experimental import pallas as pl
from jax.experimental.pallas import tpu as pltpu
from jax.experimental.pallas import tpu_sc as plsc

NUM_EMB = 1000000
DIM = 64
BATCH = 16384
HIST = 50

NC = 2
NS = 16
NW = NC * NS
GB = 128
NGB = BATCH // GB // NW
IDX_PER_BLK = GB * HIST
NGRP = NGB * HIST

_MESH = plsc.VectorSubcoreMesh(core_axis_name="c", subcore_axis_name="s")


@functools.partial(
    pl.kernel,
    mesh=_MESH,
    out_type=jax.ShapeDtypeStruct((HIST, DIM, BATCH), jnp.float32),
    scratch_types=[
        pltpu.VMEM((NGB * IDX_PER_BLK,), jnp.int32),
        pltpu.VMEM((GB,), jnp.int32),
        pltpu.VMEM((GB,), jnp.int32),
        pltpu.VMEM((GB // 16, 16), jnp.int32),
        pltpu.VMEM((GB // 16, 16), jnp.int32),
        pltpu.VMEM((GB, 128), jnp.float32),
        pltpu.VMEM((GB, 128), jnp.float32),
        pltpu.VMEM((DIM, GB), jnp.float32),
        pltpu.VMEM((DIM, GB), jnp.float32),
        pltpu.SemaphoreType.DMA,
        pltpu.SemaphoreType.DMA,
        pltpu.SemaphoreType.DMA,
        pltpu.SemaphoreType.DMA,
    ],
    compiler_params=pltpu.CompilerParams(
        use_tc_tiling_on_sc=True, needs_layout_passes=False),
)
def _emb_lookup(idx_hbm, tbl_hbm, out_hbm, stage_v, qa, qb, para, parb,
                rowsa, rowsb, dsta, dstb, g0, g1, w0, w1):
    wid = lax.axis_index("s") * NC + lax.axis_index("c")
    lanes = lax.iota(jnp.int32, 16)
    qs, pars, rows, dsts = (qa, qb), (para, parb), (rowsa, rowsb), (dsta, dstb)
    gs, ws = (g0, g1), (w0, w1)

    pltpu.sync_copy(
        idx_hbm.at[pl.ds(wid * NGB * IDX_PER_BLK, NGB * IDX_PER_BLK)],
        stage_v)

    def extract(g, p):
        base = (g // HIST) * IDX_PER_BLK + g % HIST
        for k in range(GB // 16):
            pos = (lanes + 16 * k) * HIST + base
            ids = plsc.load_gather(stage_v, [pos])
            qs[p][pl.ds(16 * k, 16)] = lax.shift_right_logical(ids, 1)
            pars[p][k, :] = lax.mul(lax.bitwise_and(ids, 1), 64)

    def fire_gather(p):
        pltpu.make_async_copy(tbl_hbm.at[qs[p]], rows[p], gs[p]).start()

    def wait_gather(p):
        pltpu.make_async_copy(tbl_hbm.at[qs[p]], rows[p], gs[p]).wait()

    def transpose(p):
        colbs = tuple(pars[p][k, :] for k in range(GB // 16))
        for f in range(DIM):
            vals = [plsc.load_gather(rows[p], [lanes + 16 * k, colbs[k] + f])
                    for k in range(GB // 16)]
            for k in range(GB // 16):
                dsts[p][f, pl.ds(16 * k, 16)] = vals[k]

    def out_slice(g):
        return out_hbm.at[g % HIST, pl.ds(0, DIM),
                          pl.ds((wid * NGB + g // HIST) * GB, GB)]

    def fire_write(g, p):
        pltpu.make_async_copy(dsts[p], out_slice(g), ws[p]).start()

    def wait_write(p):
        pltpu.make_async_copy(dsts[p], out_slice(0), ws[p]).wait()

    def phase(g, p, first=False, wait_w=True):
        extract(g, p)
        if wait_w:
            wait_write(p)
        fire_gather(p)
        if not first:
            wait_gather(1 - p)
            transpose(1 - p)
            fire_write(g - 1, 1 - p)

    phase(0, 0, first=True, wait_w=False)
    phase(1, 1, wait_w=False)

    def body(t, carry):
        phase(2 * t, 0)
        phase(2 * t + 1, 1)
        return carry

    lax.fori_loop(1, NGRP // 2, body, 0)

    wait_gather(1)
    transpose(1)
    fire_write(NGRP - 1, 1)
    wait_write(0)
    wait_write(1)


def kernel(token_ids, weight):
    idx = token_ids.astype(jnp.int32).reshape(-1)
    tbl = weight.reshape(NUM_EMB // 2, 2 * DIM)
    out_t = _emb_lookup(idx, tbl)
    return out_t.transpose(2, 0, 1)


# speedup vs baseline: 1.7811x; 1.7811x over previous
import functools

import jax
import jax.numpy as jnp
from jax import lax
from jax.experimental import pallas as pl
from jax.experimental.pallas import tpu as pltpu
from jax.experimental.pallas import tpu_sc as plsc

NUM_EMB = 1000000
DIM = 64
BATCH = 16384
HIST = 50

NC = 2
NS = 16
NW = NC * NS
GB = 128
NGB = BATCH // GB // NW
IDX_PER_BLK = GB * HIST
NGRP = NGB * HIST

_MESH = plsc.VectorSubcoreMesh(core_axis_name="c", subcore_axis_name="s")


@functools.partial(
    pl.kernel,
    mesh=_MESH,
    out_type=jax.ShapeDtypeStruct((HIST, DIM, BATCH), jnp.float32),
    scratch_types=[
        pltpu.VMEM((NGB * IDX_PER_BLK,), jnp.int32),
        pltpu.VMEM((GB,), jnp.int32),
        pltpu.VMEM((GB,), jnp.int32),
        pltpu.VMEM((GB // 16, 16), jnp.int32),
        pltpu.VMEM((GB // 16, 16), jnp.int32),
        pltpu.VMEM((GB, 128), jnp.float32),
        pltpu.VMEM((GB, 128), jnp.float32),
        pltpu.VMEM((DIM, GB), jnp.float32),
        pltpu.VMEM((DIM, GB), jnp.float32),
        pltpu.SemaphoreType.DMA,
        pltpu.SemaphoreType.DMA,
        pltpu.SemaphoreType.DMA,
        pltpu.SemaphoreType.DMA,
    ],
    compiler_params=pltpu.CompilerParams(
        use_tc_tiling_on_sc=True, needs_layout_passes=False),
)
def _emb_lookup(idx_hbm, tbl_hbm, out_hbm, stage_v, qa, qb, para, parb,
                rowsa, rowsb, dsta, dstb, g0, g1, w0, w1):
    wid = lax.axis_index("s") * NC + lax.axis_index("c")
    lanes = lax.iota(jnp.int32, 16)
    qs, pars, rows, dsts = (qa, qb), (para, parb), (rowsa, rowsb), (dsta, dstb)
    gs, ws = (g0, g1), (w0, w1)

    pltpu.sync_copy(
        idx_hbm.at[pl.ds(wid * NGB * IDX_PER_BLK, NGB * IDX_PER_BLK)],
        stage_v)

    def extract(g, p):
        base = (g // HIST) * IDX_PER_BLK + g % HIST
        for k in range(GB // 16):
            pos = (lanes + 16 * k) * HIST + base
            ids = plsc.load_gather(stage_v, [pos])
            qs[p][pl.ds(16 * k, 16)] = lax.shift_right_logical(ids, 1)
            pars[p][k, :] = lax.mul(lax.bitwise_and(ids, 1), 64)

    def fire_gather(p):
        pltpu.make_async_copy(tbl_hbm.at[qs[p]], rows[p], gs[p]).start()

    def wait_gather(p):
        pltpu.make_async_copy(tbl_hbm.at[qs[p]], rows[p], gs[p]).wait()

    rot = tuple(lax.bitwise_and(lanes + j, 15) for j in range(16))

    def transpose(p):
        def kblock(kb, carry):
            colb = pars[p][kb, :]
            rowv = lanes + 16 * kb
            for fb in range(DIM // 16):
                colbf = colb + 16 * fb
                vals = [plsc.load_gather(rows[p], [rowv, colbf + rot[j]])
                        for j in range(16)]
                for j in range(16):
                    plsc.store_scatter(
                        dsts[p], [rot[j] + 16 * fb, rowv], vals[j])
            return carry

        lax.fori_loop(0, GB // 16, kblock, 0)

    def out_slice(g):
        return out_hbm.at[g % HIST, pl.ds(0, DIM),
                          pl.ds((wid * NGB + g // HIST) * GB, GB)]

    def fire_write(g, p):
        pltpu.make_async_copy(dsts[p], out_slice(g), ws[p]).start()

    def wait_write(p):
        pltpu.make_async_copy(dsts[p], out_slice(0), ws[p]).wait()

    def phase(g, p, first=False, wait_w=True):
        extract(g, p)
        if wait_w:
            wait_write(p)
        fire_gather(p)
        if not first:
            wait_gather(1 - p)
            transpose(1 - p)
            fire_write(g - 1, 1 - p)

    phase(0, 0, first=True, wait_w=False)
    phase(1, 1, wait_w=False)

    def body(t, carry):
        phase(2 * t, 0)
        phase(2 * t + 1, 1)
        return carry

    lax.fori_loop(1, NGRP // 2, body, 0)

    wait_gather(1)
    transpose(1)
    fire_write(NGRP - 1, 1)
    wait_write(0)
    wait_write(1)


def kernel(token_ids, weight):
    idx = token_ids.astype(jnp.int32).reshape(-1)
    tbl = weight.reshape(NUM_EMB // 2, 2 * DIM)
    out_t = _emb_lookup(idx, tbl)
    return out_t.transpose(2, 0, 1)
